# baseline (speedup 1.0000x reference)
import functools

import jax
import jax.numpy as jnp
from jax import lax
from jax.experimental import pallas as pl
from jax.experimental.pallas import tpu as pltpu
from jax.experimental.pallas import tpu_sc as plsc

_ROWS = 32 * 512
_COLS = 512
_TOTAL = float(_ROWS * _COLS)

_SC_ROWS = 16384
_NW = 32
_WROWS = _SC_ROWS // _NW
_CH = 16
_CHW = _CH * _COLS
_NCHUNK = _WROWS // _CH
_LANES = 16

_TC_ROWS = _ROWS - _SC_ROWS
_BLOCK_ROWS = 1024
_GRID = _TC_ROWS // _BLOCK_ROWS
_SC_BLOCKS = _SC_ROWS // _BLOCK_ROWS
_STRIP = 32


def _sc_chunk_accum(rbuf, tbuf, carry):
    def row(i, carry):
        def vec(c, carry):
            af, at, ac = carry
            r = rbuf[i, pl.ds(c * _LANES, _LANES)]
            t = tbuf[i, pl.ds(c * _LANES, _LANES)]
            d = r - t
            d2 = d * d
            mf = t > 0.0
            af = af + jnp.where(mf, d2, 0.0)
            at = at + d2
            ac = ac + jnp.where(mf, 1.0, 0.0)
            return af, at, ac

        return lax.fori_loop(0, _COLS // _LANES, vec, carry, unroll=8)

    return lax.fori_loop(0, _CH, row, carry)


def _sc_body(reg_hbm, tgt_hbm, out_hbm, rb0, rb1, tb0, tb1, obuf, s0, s1):
    c = lax.axis_index("c")
    s = lax.axis_index("s")
    wid = s * 2 + c
    base = wid * _WROWS

    rbufs = (rb0, rb1)
    tbufs = (tb0, tb1)
    sems = (s0, s1)

    def start(k, b):
        off = base + k * _CH
        hr = pltpu.async_copy(
            reg_hbm.at[pl.ds(off, _CH), :], rbufs[b], sems[b]
        )
        ht = pltpu.async_copy(
            tgt_hbm.at[pl.ds(off, _CH), :], tbufs[b], sems[b]
        )
        return hr, ht

    zero = jnp.zeros((_LANES,), jnp.float32)
    carry = (zero, zero, zero)
    pending = start(0, 0)
    for k in range(_NCHUNK):
        b = k % 2
        hr, ht = pending
        if k + 1 < _NCHUNK:
            pending = start(k + 1, (k + 1) % 2)
        hr.wait()
        ht.wait()
        carry = _sc_chunk_accum(rbufs[b], tbufs[b], carry)

    af, at, ac = carry
    obuf[pl.ds(0, _LANES)] = af
    obuf[pl.ds(_LANES, _LANES)] = at
    obuf[pl.ds(2 * _LANES, _LANES)] = ac
    pltpu.sync_copy(obuf, out_hbm.at[wid])


_sc_partial = functools.partial(
    pl.kernel,
    out_type=jax.ShapeDtypeStruct((_NW, 3 * _LANES), jnp.float32),
    mesh=plsc.VectorSubcoreMesh(
        core_axis_name="c", subcore_axis_name="s", num_cores=2
    ),
    scratch_types=[
        pltpu.VMEM((_CH, _COLS), jnp.float32),
        pltpu.VMEM((_CH, _COLS), jnp.float32),
        pltpu.VMEM((_CH, _COLS), jnp.float32),
        pltpu.VMEM((_CH, _COLS), jnp.float32),
        pltpu.VMEM((3 * _LANES,), jnp.float32),
        pltpu.SemaphoreType.DMA,
        pltpu.SemaphoreType.DMA,
    ],
)(_sc_body)


def _tc_body(reg_ref, tgt_ref, acc_ref):
    i = pl.program_id(0)

    def strip(s, carry):
        af, at, ac = carry
        r = reg_ref[pl.ds(s * _STRIP, _STRIP), :]
        t = tgt_ref[pl.ds(s * _STRIP, _STRIP), :]
        d = r - t
        d2 = d * d
        mf = t > 0.0
        af = af + jnp.where(mf, d2, 0.0)
        at = at + d2
        ac = ac + jnp.where(mf, 1.0, 0.0)
        return af, at, ac

    zero = jnp.zeros((_STRIP, _COLS), jnp.float32)
    af, at, ac = lax.fori_loop(
        0, _BLOCK_ROWS // _STRIP, strip, (zero, zero, zero), unroll=2
    )
    fsum = jnp.sum(af)
    tsum = jnp.sum(at)
    fcnt = jnp.sum(ac)

    @pl.when(i == 0)
    def _():
        acc_ref[0] = fsum
        acc_ref[1] = tsum
        acc_ref[2] = fcnt

    @pl.when(i > 0)
    def _():
        acc_ref[0] += fsum
        acc_ref[1] += tsum
        acc_ref[2] += fcnt


@jax.jit
def _run(reg, targets):
    reg2 = reg.reshape(_ROWS, _COLS)
    tgt2 = targets.reshape(_ROWS, _COLS)

    tc_part = jnp.zeros((4,), jnp.float32)

    sc_part = _sc_partial(reg2, tgt2)

    p = sc_part.reshape(_NW, 3, _LANES)
    sf = tc_part[0] + jnp.sum(p[:, 0, :])
    st = tc_part[1] + jnp.sum(p[:, 1, :])
    nf = tc_part[2] + jnp.sum(p[:, 2, :])
    su = st - sf
    nu = _TOTAL - nf
    flood = jnp.where(nf > 0.0, sf / jnp.maximum(nf, 1.0), 0.0)
    unflood = jnp.where(nu > 0.0, su / jnp.maximum(nu, 1.0), 0.0)
    loss_reg = 20.0 * flood + unflood
    loss_cls = jnp.zeros(1, dtype=jnp.float32)
    loss = 2.0 * loss_reg + loss_cls
    return (
        loss,
        2.0 * loss_reg,
        2.0 * flood,
        2.0 * unflood,
        loss_reg,
        flood,
        unflood,
        loss_cls,
    )


def kernel(reg, targets):
    return _run(reg, targets)


# speedup vs baseline: 3.0600x; 3.0600x over previous
import jax
import jax.numpy as jnp
from jax import lax
from jax.experimental import pallas as pl
from jax.experimental.pallas import tpu as pltpu

_ROWS = 32 * 512
_COLS = 512
_BLOCK_ROWS = 2048
_GRID = _ROWS // _BLOCK_ROWS
_STRIP = 32
_TOTAL = float(_ROWS * _COLS)


def _body(reg_ref, tgt_ref, o0, o1, o2, o3, o4, o5, o6, o7, acc_ref):
    i = pl.program_id(0)

    def strip(s, carry):
        af, at, ac = carry
        r = reg_ref[pl.ds(s * _STRIP, _STRIP), :]
        t = tgt_ref[pl.ds(s * _STRIP, _STRIP), :]
        d = r - t
        d2 = d * d
        mf = t > 0.0
        af = af + jnp.where(mf, d2, 0.0)
        at = at + d2
        ac = ac + jnp.where(mf, 1.0, 0.0)
        return af, at, ac

    zero = jnp.zeros((_STRIP, _COLS), jnp.float32)
    af, at, ac = lax.fori_loop(
        0, _BLOCK_ROWS // _STRIP, strip, (zero, zero, zero), unroll=2
    )
    fsum = jnp.sum(af)
    tsum = jnp.sum(at)
    fcnt = jnp.sum(ac)

    @pl.when(i == 0)
    def _():
        acc_ref[0] = fsum
        acc_ref[1] = tsum
        acc_ref[2] = fcnt

    @pl.when(i > 0)
    def _():
        acc_ref[0] += fsum
        acc_ref[1] += tsum
        acc_ref[2] += fcnt

    @pl.when(i == _GRID - 1)
    def _():
        sf = acc_ref[0]
        st = acc_ref[1]
        nf = acc_ref[2]
        su = st - sf
        nu = _TOTAL - nf
        flood = jnp.where(nf > 0.0, sf / jnp.maximum(nf, 1.0), 0.0)
        unflood = jnp.where(nu > 0.0, su / jnp.maximum(nu, 1.0), 0.0)
        loss_reg = 20.0 * flood + unflood
        o0[0] = 2.0 * loss_reg
        o1[0] = 2.0 * loss_reg
        o2[0] = 2.0 * flood
        o3[0] = 2.0 * unflood
        o4[0] = loss_reg
        o5[0] = flood
        o6[0] = unflood
        o7[0] = 0.0


@jax.jit
def _run(reg, targets):
    reg2 = reg.reshape(_ROWS, _COLS)
    tgt2 = targets.reshape(_ROWS, _COLS)
    sds = jax.ShapeDtypeStruct((1,), jnp.float32)
    outs = pl.pallas_call(
        _body,
        grid=(_GRID,),
        in_specs=[
            pl.BlockSpec((_BLOCK_ROWS, _COLS), lambda i: (i, 0)),
            pl.BlockSpec((_BLOCK_ROWS, _COLS), lambda i: (i, 0)),
        ],
        out_specs=[pl.BlockSpec(memory_space=pltpu.SMEM)] * 8,
        out_shape=[sds] * 8,
        scratch_shapes=[pltpu.SMEM((4,), jnp.float32)],
        compiler_params=pltpu.CompilerParams(
            dimension_semantics=("arbitrary",)
        ),
    )(reg2, tgt2)
    return (
        outs[0],
        outs[1].reshape(()),
        outs[2].reshape(()),
        outs[3].reshape(()),
        outs[4].reshape(()),
        outs[5].reshape(()),
        outs[6].reshape(()),
        outs[7],
    )


def kernel(reg, targets):
    return _run(reg, targets)
